# Initial kernel scaffold; baseline (speedup 1.0000x reference)
import jax
import jax.numpy as jnp
from jax.experimental import pallas as pl


def kernel(embedding, sub_ind, obj_ind, W1, b1, W2, b2, W3, b3):
    raise NotImplementedError("write your pallas kernel here")


# speedup vs baseline: 2.8541x; 2.8541x over previous
import functools

import jax
import jax.numpy as jnp
from jax import lax
from jax.experimental import pallas as pl
from jax.experimental.pallas import tpu as pltpu
from jax.experimental.pallas import tpu_sc as plsc

B, C, H, W = 16, 256, 128, 128
HW = H * W
K = 128
HIDDEN = 1024
NUM_CLASSES = 117
LANES = 16
GRAN_ROWS = HW // LANES
N_PAIRS = K
N_WORKERS = 32


LAG = 4


def _sc_gather_kernel(table_hbm, inds_hbm, out_hbm, ind_v, idx_v, out_v, sem):
    wid = lax.axis_index("c") * 16 + lax.axis_index("s")
    b = lax.rem(wid, 16)
    base = b * (C * HW)
    pltpu.sync_copy(inds_hbm.at[pl.ds(wid * N_PAIRS, N_PAIRS)], ind_v)

    def fire(j, p):
        for c16 in range(C // LANES):
            cvec = (lax.iota(jnp.int32, LANES) + (c16 * LANES)) * HW
            idx_v[pl.ds(j * C + c16 * LANES, LANES)] = cvec + (p + base)
        pltpu.async_copy(table_hbm.at[idx_v.at[pl.ds(j * C, 128)]],
                         out_v.at[pl.ds(j * C, 128)], sem)
        pltpu.async_copy(table_hbm.at[idx_v.at[pl.ds(j * C + 128, 128)]],
                         out_v.at[pl.ds(j * C + 128, 128)], sem)

    def drain(j):
        pltpu.make_async_copy(table_hbm.at[idx_v.at[pl.ds(j * C, 128)]],
                              out_v.at[pl.ds(j * C, 128)], sem).wait()
        pltpu.make_async_copy(table_hbm.at[idx_v.at[pl.ds(j * C + 128, 128)]],
                              out_v.at[pl.ds(j * C + 128, 128)], sem).wait()

    def group_body(g, carry):
        p_vec = ind_v[pl.ds(g * LANES, LANES)]
        for j16 in range(LANES):
            fire(g * LANES + j16, p_vec[j16])

        @pl.when(g >= 1)
        def _():
            for j16 in range(LANES):
                drain((g - 1) * LANES + j16)

        return carry

    lax.fori_loop(0, N_PAIRS // LANES, group_body, None)
    for j in range(N_PAIRS - LANES, N_PAIRS):
        drain(j)
    pltpu.sync_copy(out_v, out_hbm.at[pl.ds(wid * (N_PAIRS * C), N_PAIRS * C)])


def _mlp_body(sub_ref, obj_ref, w1a_ref, w1b_ref, b1_ref, w2_ref, b2_ref,
              w3_ref, b3_ref, out_ref):
    dn = (((1,), (1,)), ((), ()))
    x = lax.dot_general(sub_ref[...], w1a_ref[...], dn,
                        preferred_element_type=jnp.float32)
    x = x + lax.dot_general(obj_ref[...], w1b_ref[...], dn,
                            preferred_element_type=jnp.float32)
    h1 = jnp.maximum(x + b1_ref[...], 0.0)
    h2 = jnp.maximum(
        lax.dot_general(h1, w2_ref[...], dn,
                        preferred_element_type=jnp.float32) + b2_ref[...], 0.0)
    out_ref[...] = lax.dot_general(
        h2, w3_ref[...], dn, preferred_element_type=jnp.float32) + b3_ref[...]


def kernel(embedding, sub_ind, obj_ind, W1, b1, W2, b2, W3, b3):
    table = embedding.reshape(B * C * HW)
    inds = jnp.concatenate(
        [sub_ind.reshape(-1).astype(jnp.int32),
         obj_ind.reshape(-1).astype(jnp.int32)])

    mesh = plsc.VectorSubcoreMesh(core_axis_name="c", subcore_axis_name="s")
    gathered = pl.kernel(
        _sc_gather_kernel,
        mesh=mesh,
        out_type=jax.ShapeDtypeStruct((2 * B * K * C,), jnp.float32),
        scratch_types=[
            pltpu.VMEM((N_PAIRS,), jnp.int32),
            pltpu.VMEM((N_PAIRS * C,), jnp.int32),
            pltpu.VMEM((N_PAIRS * C,), jnp.float32),
            pltpu.SemaphoreType.DMA,
        ],
    )(table, inds)

    feats = gathered.reshape(2, B * K, C)
    sub_feat, obj_feat = feats[0], feats[1]

    W1a = W1[:, :C]
    W1b = W1[:, C:]
    W3p = jnp.zeros((128, HIDDEN), jnp.float32).at[:NUM_CLASSES].set(W3)
    b3p = jnp.zeros((1, 128), jnp.float32).at[0, :NUM_CLASSES].set(b3)

    m_tile = 512
    out = pl.pallas_call(
        _mlp_body,
        grid=(B * K // m_tile,),
        in_specs=[
            pl.BlockSpec((m_tile, C), lambda m: (m, 0)),
            pl.BlockSpec((m_tile, C), lambda m: (m, 0)),
            pl.BlockSpec((HIDDEN, C), lambda m: (0, 0)),
            pl.BlockSpec((HIDDEN, C), lambda m: (0, 0)),
            pl.BlockSpec((1, HIDDEN), lambda m: (0, 0)),
            pl.BlockSpec((HIDDEN, HIDDEN), lambda m: (0, 0)),
            pl.BlockSpec((1, HIDDEN), lambda m: (0, 0)),
            pl.BlockSpec((128, HIDDEN), lambda m: (0, 0)),
            pl.BlockSpec((1, 128), lambda m: (0, 0)),
        ],
        out_specs=pl.BlockSpec((m_tile, 128), lambda m: (m, 0)),
        out_shape=jax.ShapeDtypeStruct((B * K, 128), jnp.float32),
    )(sub_feat, obj_feat, W1a, W1b, b1.reshape(1, HIDDEN), W2,
      b2.reshape(1, HIDDEN), W3p, b3p)

    return out[:, :NUM_CLASSES].reshape(B, K, NUM_CLASSES)


# speedup vs baseline: 2.9699x; 1.0406x over previous
import functools

import jax
import jax.numpy as jnp
from jax import lax
from jax.experimental import pallas as pl
from jax.experimental.pallas import tpu as pltpu
from jax.experimental.pallas import tpu_sc as plsc

B, C, H, W = 16, 256, 128, 128
HW = H * W
K = 128
HIDDEN = 1024
NUM_CLASSES = 117
LANES = 16
GRAN_ROWS = HW // LANES
N_PAIRS = K
N_WORKERS = 32


LAG = 4


def _sc_gather_kernel(table_hbm, inds_hbm, out_hbm, ind_v, idx_v, out_v, sem):
    wid = lax.axis_index("c") * 16 + lax.axis_index("s")
    b = lax.rem(wid, 16)
    base = b * (C * HW)
    pltpu.sync_copy(inds_hbm.at[pl.ds(wid * N_PAIRS, N_PAIRS)], ind_v)

    def fire(j, p):
        for c16 in range(C // LANES):
            cvec = (lax.iota(jnp.int32, LANES) + (c16 * LANES)) * HW
            idx_v[pl.ds(j * C + c16 * LANES, LANES)] = cvec + (p + base)
        pltpu.async_copy(table_hbm.at[idx_v.at[pl.ds(j * C, 128)]],
                         out_v.at[pl.ds(j * C, 128)], sem)
        pltpu.async_copy(table_hbm.at[idx_v.at[pl.ds(j * C + 128, 128)]],
                         out_v.at[pl.ds(j * C + 128, 128)], sem)

    def drain(j):
        pltpu.make_async_copy(table_hbm.at[idx_v.at[pl.ds(j * C, 128)]],
                              out_v.at[pl.ds(j * C, 128)], sem).wait()
        pltpu.make_async_copy(table_hbm.at[idx_v.at[pl.ds(j * C + 128, 128)]],
                              out_v.at[pl.ds(j * C + 128, 128)], sem).wait()

    def group_body(g, carry):
        p_vec = ind_v[pl.ds(g * LANES, LANES)]
        for j16 in range(LANES):
            fire(g * LANES + j16, p_vec[j16])

        @pl.when(g >= 1)
        def _():
            for j16 in range(LANES):
                drain((g - 1) * LANES + j16)

        return carry

    lax.fori_loop(0, N_PAIRS // LANES, group_body, None)
    for j in range(N_PAIRS - LANES, N_PAIRS):
        drain(j)
    pltpu.sync_copy(out_v, out_hbm.at[pl.ds(wid * (N_PAIRS * C), N_PAIRS * C)])


def _mlp_body(sub_ref, obj_ref, w1a_ref, w1b_ref, b1_ref, w2_ref, b2_ref,
              w3_ref, b3_ref, out_ref):
    dn = (((1,), (1,)), ((), ()))
    x = lax.dot_general(sub_ref[...], w1a_ref[...], dn,
                        preferred_element_type=jnp.float32)
    x = x + lax.dot_general(obj_ref[...], w1b_ref[...], dn,
                            preferred_element_type=jnp.float32)
    h1 = jnp.maximum(x + b1_ref[...], 0.0)
    h2 = jnp.maximum(
        lax.dot_general(h1, w2_ref[...], dn,
                        preferred_element_type=jnp.float32) + b2_ref[...], 0.0)
    out_ref[...] = lax.dot_general(
        h2, w3_ref[...], dn, preferred_element_type=jnp.float32) + b3_ref[...]


def kernel(embedding, sub_ind, obj_ind, W1, b1, W2, b2, W3, b3):
    table = embedding.reshape(B * C * HW)
    inds = jnp.concatenate(
        [sub_ind.reshape(-1).astype(jnp.int32),
         obj_ind.reshape(-1).astype(jnp.int32)])

    mesh = plsc.VectorSubcoreMesh(core_axis_name="c", subcore_axis_name="s")
    gathered = pl.kernel(
        _sc_gather_kernel,
        mesh=mesh,
        out_type=jax.ShapeDtypeStruct((2 * B * K * C,), jnp.float32),
        scratch_types=[
            pltpu.VMEM((N_PAIRS,), jnp.int32),
            pltpu.VMEM((N_PAIRS * C,), jnp.int32),
            pltpu.VMEM((N_PAIRS * C,), jnp.float32),
            pltpu.SemaphoreType.DMA,
        ],
    )(table, inds)

    feats = gathered.reshape(2 * B * K, C)
    n_sub_blocks = B * K // 512

    m_tile = 512
    out = pl.pallas_call(
        _mlp_body,
        grid=(B * K // m_tile,),
        in_specs=[
            pl.BlockSpec((m_tile, C), lambda m: (m, 0)),
            pl.BlockSpec((m_tile, C), lambda m: (m + n_sub_blocks, 0)),
            pl.BlockSpec((HIDDEN, C), lambda m: (0, 0)),
            pl.BlockSpec((HIDDEN, C), lambda m: (0, 1)),
            pl.BlockSpec((1, HIDDEN), lambda m: (0, 0)),
            pl.BlockSpec((HIDDEN, HIDDEN), lambda m: (0, 0)),
            pl.BlockSpec((1, HIDDEN), lambda m: (0, 0)),
            pl.BlockSpec((NUM_CLASSES, HIDDEN), lambda m: (0, 0)),
            pl.BlockSpec((1, NUM_CLASSES), lambda m: (0, 0)),
        ],
        out_specs=pl.BlockSpec((m_tile, NUM_CLASSES), lambda m: (m, 0)),
        out_shape=jax.ShapeDtypeStruct((B * K, NUM_CLASSES), jnp.float32),
    )(feats, feats, W1, W1, b1.reshape(1, HIDDEN), W2,
      b2.reshape(1, HIDDEN), W3, b3.reshape(1, NUM_CLASSES))

    return out.reshape(B, K, NUM_CLASSES)


# speedup vs baseline: 3.1690x; 1.0670x over previous
import functools

import jax
import jax.numpy as jnp
from jax import lax
from jax.experimental import pallas as pl
from jax.experimental.pallas import tpu as pltpu
from jax.experimental.pallas import tpu_sc as plsc

B, C, H, W = 16, 256, 128, 128
HW = H * W
K = 128
HIDDEN = 1024
NUM_CLASSES = 117
LANES = 16
GRAN_ROWS = HW // LANES
N_PAIRS = K
N_WORKERS = 32


LAG = 4


def _sc_gather_kernel(table_hbm, inds_hbm, out_hbm, ind_v, idx_v, out_v, sem):
    wid = lax.axis_index("c") * 16 + lax.axis_index("s")
    b = lax.rem(wid, 16)
    base = b * (C * HW)
    pltpu.sync_copy(inds_hbm.at[pl.ds(wid * N_PAIRS, N_PAIRS)], ind_v)

    def fire(j, p):
        for c16 in range(C // LANES):
            cvec = (lax.iota(jnp.int32, LANES) + (c16 * LANES)) * HW
            idx_v[pl.ds(j * C + c16 * LANES, LANES)] = cvec + (p + base)
        pltpu.async_copy(table_hbm.at[idx_v.at[pl.ds(j * C, 128)]],
                         out_v.at[j, pl.ds(0, 128)], sem)
        pltpu.async_copy(table_hbm.at[idx_v.at[pl.ds(j * C + 128, 128)]],
                         out_v.at[j, pl.ds(128, 128)], sem)

    def drain(j):
        pltpu.make_async_copy(table_hbm.at[idx_v.at[pl.ds(j * C, 128)]],
                              out_v.at[j, pl.ds(0, 128)], sem).wait()
        pltpu.make_async_copy(table_hbm.at[idx_v.at[pl.ds(j * C + 128, 128)]],
                              out_v.at[j, pl.ds(128, 128)], sem).wait()

    def group_body(g, carry):
        p_vec = ind_v[pl.ds(g * LANES, LANES)]
        for j16 in range(LANES):
            fire(g * LANES + j16, p_vec[j16])

        @pl.when(g >= 1)
        def _():
            for j16 in range(LANES):
                drain((g - 1) * LANES + j16)

        return carry

    lax.fori_loop(0, N_PAIRS // LANES, group_body, None)
    for j in range(N_PAIRS - LANES, N_PAIRS):
        drain(j)
    pltpu.sync_copy(out_v, out_hbm.at[pl.ds(wid * N_PAIRS, N_PAIRS), :])


def _mlp_body(sub_ref, obj_ref, w1a_ref, w1b_ref, b1_ref, w2_ref, b2_ref,
              w3_ref, b3_ref, out_ref):
    dn = (((1,), (1,)), ((), ()))
    x = lax.dot_general(sub_ref[...], w1a_ref[...], dn,
                        preferred_element_type=jnp.float32)
    x = x + lax.dot_general(obj_ref[...], w1b_ref[...], dn,
                            preferred_element_type=jnp.float32)
    h1 = jnp.maximum(x + b1_ref[...], 0.0)
    h2 = jnp.maximum(
        lax.dot_general(h1, w2_ref[...], dn,
                        preferred_element_type=jnp.float32) + b2_ref[...], 0.0)
    out = lax.dot_general(
        h2, w3_ref[...], dn, preferred_element_type=jnp.float32) + b3_ref[...]
    out_ref[...] = out.reshape(out_ref.shape)


def kernel(embedding, sub_ind, obj_ind, W1, b1, W2, b2, W3, b3):
    table = embedding.reshape(B * C * HW)
    inds = jnp.concatenate(
        [sub_ind.reshape(-1).astype(jnp.int32),
         obj_ind.reshape(-1).astype(jnp.int32)])

    mesh = plsc.VectorSubcoreMesh(core_axis_name="c", subcore_axis_name="s")
    gathered = pl.kernel(
        _sc_gather_kernel,
        mesh=mesh,
        out_type=jax.ShapeDtypeStruct((2 * B * K, C), jnp.float32),
        scratch_types=[
            pltpu.VMEM((N_PAIRS,), jnp.int32),
            pltpu.VMEM((N_PAIRS * C,), jnp.int32),
            pltpu.VMEM((N_PAIRS, C), jnp.float32),
            pltpu.SemaphoreType.DMA,
        ],
    )(table, inds)

    feats = gathered
    n_sub_blocks = B * K // 512

    m_tile = 512
    out = pl.pallas_call(
        _mlp_body,
        grid=(B * K // m_tile,),
        in_specs=[
            pl.BlockSpec((m_tile, C), lambda m: (m, 0)),
            pl.BlockSpec((m_tile, C), lambda m: (m + n_sub_blocks, 0)),
            pl.BlockSpec((HIDDEN, C), lambda m: (0, 0)),
            pl.BlockSpec((HIDDEN, C), lambda m: (0, 1)),
            pl.BlockSpec((1, HIDDEN), lambda m: (0, 0)),
            pl.BlockSpec((HIDDEN, HIDDEN), lambda m: (0, 0)),
            pl.BlockSpec((1, HIDDEN), lambda m: (0, 0)),
            pl.BlockSpec((NUM_CLASSES, HIDDEN), lambda m: (0, 0)),
            pl.BlockSpec((1, NUM_CLASSES), lambda m: (0, 0)),
        ],
        out_specs=pl.BlockSpec((m_tile // K, K, NUM_CLASSES),
                               lambda m: (m, 0, 0)),
        out_shape=jax.ShapeDtypeStruct((B, K, NUM_CLASSES), jnp.float32),
    )(feats, feats, W1, W1, b1.reshape(1, HIDDEN), W2,
      b2.reshape(1, HIDDEN), W3, b3.reshape(1, NUM_CLASSES))

    return out
